# baseline (speedup 1.0000x reference)
import jax
import jax.numpy as jnp
from jax import lax
from jax.experimental import pallas as pl
from jax.experimental.pallas import tpu as pltpu
from jax.experimental.pallas import tpu_sc as plsc

IN_N = 4096
OUT_N = 1024
DELAYS = 16
V_TH = 1.0
TAU = 5.0

SC_ROWS = 256
NW = 32
ROWS_PER_W = SC_ROWS // NW
TC_ROWS = OUT_N - SC_ROWS

BJ = 32
NSTEP_TC = TC_ROWS // BJ

HALF_K = DELAYS // 2
HALF_I = IN_N // 2
CSTEPS = HALF_I // 16


def _o_body(t_ref, x_ref, d_ref, o_ref):
    tval = t_ref[0, 0]
    xx = x_ref[...]
    tt = tval - xx - d_ref[...]
    mask = jnp.logical_and(xx != -1.0, tt >= 0.0)
    o_ref[...] = jnp.where(mask, tt * jnp.exp(1.0 - tt / TAU) / TAU, 0.0)


def _tc_body(t_ref, s_ref, o_ref, w_ref, out_ref):
    tval = t_ref[0, 0]
    prod = w_ref[...] * o_ref[...][None]
    v = jnp.sum(prod, axis=(1, 2))
    s_old = s_ref[...]
    fire = jnp.logical_and(s_old < 0.0, v[:, None] >= V_TH)
    out_ref[...] = jnp.where(fire, tval, s_old)


def _sel_body(t_ref, s_ref, v_ref, out_ref):
    tval = t_ref[0, 0]
    s_old = s_ref[...]
    fire = jnp.logical_and(s_old < 0.0, v_ref[...] >= V_TH)
    out_ref[...] = jnp.where(fire, tval, s_old)


def _sc_body(w_hbm, o_hbm, out_hbm, o_v, rv0, rv1, vout_v, sem0, sem1):
    wid = lax.axis_index("s") * 2 + lax.axis_index("c")
    base = TC_ROWS + wid * ROWS_PER_W
    pltpu.sync_copy(o_hbm, o_v)

    @pl.loop(0, ROWS_PER_W)
    def _(r):
        j = base + r
        rvs = (rv0, rv1)
        sems = (sem0, sem1)

        def chunk_src(m):
            h, half = divmod(m, 2)
            return w_hbm.at[j, pl.ds(h * HALF_K, HALF_K),
                            pl.ds(half * HALF_I, HALF_I)]

        cps = {0: pltpu.async_copy(chunk_src(0), rv0, sem0)}
        acc = jnp.zeros((16,), jnp.float32)
        for m in range(4):
            if m < 3:
                nb = (m + 1) % 2
                cps[m + 1] = pltpu.async_copy(chunk_src(m + 1), rvs[nb], sems[nb])
            cps[m].wait()
            rv = rvs[m % 2]
            h, half = divmod(m, 2)
            for kp in range(HALF_K):
                def mac(c, a, _kp=kp, _h=h, _half=half, _rv=rv):
                    return a + (_rv[_kp, pl.ds(c * 16, 16)]
                                * o_v[_h * HALF_K + _kp,
                                      pl.ds(_half * HALF_I + c * 16, 16)])
                acc = lax.fori_loop(0, CSTEPS, mac, acc)
        vout_v[r] = jnp.full((16,), jnp.sum(acc), jnp.float32)

    pltpu.sync_copy(vout_v, out_hbm.at[wid])


def kernel(t, x, w, d, s):
    wt = jnp.transpose(w, (0, 2, 1))
    x2 = x.reshape(1, IN_N)
    d2 = d.reshape(DELAYS, 1)
    t2 = jnp.asarray(t, jnp.float32).reshape(1, 1)
    s2 = s.reshape(OUT_N, 1)

    o_plane = pl.pallas_call(
        _o_body,
        in_specs=[
            pl.BlockSpec((1, 1), lambda: (0, 0)),
            pl.BlockSpec((1, IN_N), lambda: (0, 0)),
            pl.BlockSpec((DELAYS, 1), lambda: (0, 0)),
        ],
        out_specs=pl.BlockSpec((DELAYS, IN_N), lambda: (0, 0)),
        out_shape=jax.ShapeDtypeStruct((DELAYS, IN_N), jnp.float32),
    )(t2, x2, d2)

    out_tc = pl.pallas_call(
        _tc_body,
        grid=(NSTEP_TC,),
        in_specs=[
            pl.BlockSpec((1, 1), lambda c: (0, 0)),
            pl.BlockSpec((BJ, 1), lambda c: (c, 0)),
            pl.BlockSpec((DELAYS, IN_N), lambda c: (0, 0)),
            pl.BlockSpec((BJ, DELAYS, IN_N), lambda c: (c, 0, 0)),
        ],
        out_specs=pl.BlockSpec((BJ, 1), lambda c: (c, 0)),
        out_shape=jax.ShapeDtypeStruct((TC_ROWS, 1), jnp.float32),
    )(t2, s2, o_plane, wt)

    mesh = plsc.VectorSubcoreMesh(core_axis_name="c", subcore_axis_name="s")
    sck = pl.kernel(
        _sc_body,
        mesh=mesh,
        out_type=jax.ShapeDtypeStruct((NW, ROWS_PER_W, 16), jnp.float32),
        scratch_types=[
            pltpu.VMEM((DELAYS, IN_N), jnp.float32),
            pltpu.VMEM((HALF_K, HALF_I), jnp.float32),
            pltpu.VMEM((HALF_K, HALF_I), jnp.float32),
            pltpu.VMEM((ROWS_PER_W, 16), jnp.float32),
            pltpu.SemaphoreType.DMA,
            pltpu.SemaphoreType.DMA,
        ],
        compiler_params=pltpu.CompilerParams(use_tc_tiling_on_sc=True,
                                             needs_layout_passes=False),
    )
    v_pad = sck(wt, o_plane)
    v_sc = v_pad[:, :, 0].reshape(SC_ROWS, 1)

    out_sc = pl.pallas_call(
        _sel_body,
        in_specs=[
            pl.BlockSpec((1, 1), lambda: (0, 0)),
            pl.BlockSpec((SC_ROWS, 1), lambda: (0, 0)),
            pl.BlockSpec((SC_ROWS, 1), lambda: (0, 0)),
        ],
        out_specs=pl.BlockSpec((SC_ROWS, 1), lambda: (0, 0)),
        out_shape=jax.ShapeDtypeStruct((SC_ROWS, 1), jnp.float32),
    )(t2, s2[TC_ROWS:], v_sc)

    return jnp.concatenate([out_tc.reshape(TC_ROWS), out_sc.reshape(SC_ROWS)])


# speedup vs baseline: 1.4603x; 1.4603x over previous
import jax
import jax.numpy as jnp
from jax import lax
from jax.experimental import pallas as pl
from jax.experimental.pallas import tpu as pltpu
from jax.experimental.pallas import tpu_sc as plsc

IN_N = 4096
OUT_N = 1024
DELAYS = 16
V_TH = 1.0
TAU = 5.0

SC_ROWS = 256
NW = 32
ROWS_PER_W = SC_ROWS // NW
TC_ROWS = OUT_N - SC_ROWS

BJ = 32
NSTEP_TC = TC_ROWS // BJ

HALF_K = DELAYS // 2
HALF_I = IN_N // 2
CSTEPS = HALF_I // 16


def _o_body(t_ref, x_ref, d_ref, o_ref):
    tval = t_ref[0, 0]
    xx = x_ref[...]
    tt = tval - xx - d_ref[...]
    mask = jnp.logical_and(xx != -1.0, tt >= 0.0)
    o_ref[...] = jnp.where(mask, tt * jnp.exp(1.0 - tt / TAU) / TAU, 0.0)


def _tc_body(t_ref, s_ref, o_ref, w_ref, out_ref):
    tval = t_ref[0, 0]
    prod = w_ref[...] * o_ref[...][None]
    v = jnp.sum(prod, axis=(1, 2))
    s_old = s_ref[...]
    fire = jnp.logical_and(s_old < 0.0, v[:, None] >= V_TH)
    out_ref[...] = jnp.where(fire, tval, s_old)


def _sel_body(t_ref, s_ref, v_ref, out_ref):
    tval = t_ref[0, 0]
    s_old = s_ref[...]
    fire = jnp.logical_and(s_old < 0.0, v_ref[...] >= V_TH)
    out_ref[...] = jnp.where(fire, tval, s_old)


def _sc_body(w_hbm, o_hbm, out_hbm, o_v, rv0, rv1, vout_v, sem0, sem1):
    wid = lax.axis_index("s") * 2 + lax.axis_index("c")
    base = TC_ROWS + wid * ROWS_PER_W
    pltpu.sync_copy(o_hbm, o_v)

    @pl.loop(0, ROWS_PER_W)
    def _(r):
        j = base + r
        rvs = (rv0, rv1)
        sems = (sem0, sem1)

        def chunk_src(m):
            h, half = divmod(m, 2)
            return w_hbm.at[j, pl.ds(h * HALF_K, HALF_K),
                            pl.ds(half * HALF_I, HALF_I)]

        cps = {0: pltpu.async_copy(chunk_src(0), rv0, sem0)}
        zero = jnp.zeros((16,), jnp.float32)
        accs = (zero, zero, zero, zero)
        for m in range(4):
            if m < 3:
                nb = (m + 1) % 2
                cps[m + 1] = pltpu.async_copy(chunk_src(m + 1), rvs[nb], sems[nb])
            cps[m].wait()
            rv = rvs[m % 2]
            h, half = divmod(m, 2)
            for kp in range(HALF_K):
                def mac(i, a, _kp=kp, _ko=h * HALF_K + kp,
                        _off=half * HALF_I, _rv=rv):
                    return tuple(
                        a[u] + (_rv[_kp, pl.ds(i + u * 16, 16)]
                                * o_v[_ko, pl.ds(_off + i + u * 16, 16)])
                        for u in range(4)
                    )
                accs = plsc.parallel_loop(0, HALF_I, 64, unroll=2,
                                          carry=accs)(mac)
        vtot = (accs[0] + accs[1]) + (accs[2] + accs[3])
        vout_v[r] = jnp.full((16,), jnp.sum(vtot), jnp.float32)

    pltpu.sync_copy(vout_v, out_hbm.at[wid])


def kernel(t, x, w, d, s):
    wt = jnp.transpose(w, (0, 2, 1))
    x2 = x.reshape(1, IN_N)
    d2 = d.reshape(DELAYS, 1)
    t2 = jnp.asarray(t, jnp.float32).reshape(1, 1)
    s2 = s.reshape(OUT_N, 1)

    o_plane = pl.pallas_call(
        _o_body,
        in_specs=[
            pl.BlockSpec((1, 1), lambda: (0, 0)),
            pl.BlockSpec((1, IN_N), lambda: (0, 0)),
            pl.BlockSpec((DELAYS, 1), lambda: (0, 0)),
        ],
        out_specs=pl.BlockSpec((DELAYS, IN_N), lambda: (0, 0)),
        out_shape=jax.ShapeDtypeStruct((DELAYS, IN_N), jnp.float32),
    )(t2, x2, d2)

    out_tc = pl.pallas_call(
        _tc_body,
        grid=(NSTEP_TC,),
        in_specs=[
            pl.BlockSpec((1, 1), lambda c: (0, 0)),
            pl.BlockSpec((BJ, 1), lambda c: (c, 0)),
            pl.BlockSpec((DELAYS, IN_N), lambda c: (0, 0)),
            pl.BlockSpec((BJ, DELAYS, IN_N), lambda c: (c, 0, 0)),
        ],
        out_specs=pl.BlockSpec((BJ, 1), lambda c: (c, 0)),
        out_shape=jax.ShapeDtypeStruct((TC_ROWS, 1), jnp.float32),
    )(t2, s2, o_plane, wt)

    mesh = plsc.VectorSubcoreMesh(core_axis_name="c", subcore_axis_name="s")
    sck = pl.kernel(
        _sc_body,
        mesh=mesh,
        out_type=jax.ShapeDtypeStruct((NW, ROWS_PER_W, 16), jnp.float32),
        scratch_types=[
            pltpu.VMEM((DELAYS, IN_N), jnp.float32),
            pltpu.VMEM((HALF_K, HALF_I), jnp.float32),
            pltpu.VMEM((HALF_K, HALF_I), jnp.float32),
            pltpu.VMEM((ROWS_PER_W, 16), jnp.float32),
            pltpu.SemaphoreType.DMA,
            pltpu.SemaphoreType.DMA,
        ],
        compiler_params=pltpu.CompilerParams(use_tc_tiling_on_sc=True,
                                             needs_layout_passes=False),
    )
    v_pad = sck(wt, o_plane)
    v_sc = v_pad[:, :, 0].reshape(SC_ROWS, 1)

    out_sc = pl.pallas_call(
        _sel_body,
        in_specs=[
            pl.BlockSpec((1, 1), lambda: (0, 0)),
            pl.BlockSpec((SC_ROWS, 1), lambda: (0, 0)),
            pl.BlockSpec((SC_ROWS, 1), lambda: (0, 0)),
        ],
        out_specs=pl.BlockSpec((SC_ROWS, 1), lambda: (0, 0)),
        out_shape=jax.ShapeDtypeStruct((SC_ROWS, 1), jnp.float32),
    )(t2, s2[TC_ROWS:], v_sc)

    return jnp.concatenate([out_tc.reshape(TC_ROWS), out_sc.reshape(SC_ROWS)])


# speedup vs baseline: 1.4690x; 1.0060x over previous
import jax
import jax.numpy as jnp
from jax import lax
from jax.experimental import pallas as pl
from jax.experimental.pallas import tpu as pltpu
from jax.experimental.pallas import tpu_sc as plsc

IN_N = 4096
OUT_N = 1024
DELAYS = 16
V_TH = 1.0
TAU = 5.0

SC_ROWS = 160
NW = 32
ROWS_PER_W = SC_ROWS // NW
TC_ROWS = OUT_N - SC_ROWS

BJ = 32
NSTEP_TC = TC_ROWS // BJ

HALF_K = DELAYS // 2
HALF_I = IN_N // 2
CSTEPS = HALF_I // 16


def _o_body(t_ref, x_ref, d_ref, o_ref):
    tval = t_ref[0, 0]
    xx = x_ref[...]
    tt = tval - xx - d_ref[...]
    mask = jnp.logical_and(xx != -1.0, tt >= 0.0)
    o_ref[...] = jnp.where(mask, tt * jnp.exp(1.0 - tt / TAU) / TAU, 0.0)


def _tc_body(t_ref, s_ref, o_ref, w_ref, out_ref):
    tval = t_ref[0, 0]
    prod = w_ref[...] * o_ref[...][None]
    v = jnp.sum(prod, axis=(1, 2))
    s_old = s_ref[...]
    fire = jnp.logical_and(s_old < 0.0, v[:, None] >= V_TH)
    out_ref[...] = jnp.where(fire, tval, s_old)


def _sel_body(t_ref, s_ref, v_ref, out_ref):
    tval = t_ref[0, 0]
    s_old = s_ref[...]
    fire = jnp.logical_and(s_old < 0.0, v_ref[...] >= V_TH)
    out_ref[...] = jnp.where(fire, tval, s_old)


def _sc_body(w_hbm, o_hbm, out_hbm, o_v, rv0, rv1, vout_v, sem0, sem1):
    wid = lax.axis_index("s") * 2 + lax.axis_index("c")
    base = TC_ROWS + wid * ROWS_PER_W
    pltpu.sync_copy(o_hbm, o_v)

    @pl.loop(0, ROWS_PER_W)
    def _(r):
        j = base + r
        rvs = (rv0, rv1)
        sems = (sem0, sem1)

        def chunk_src(m):
            h, half = divmod(m, 2)
            return w_hbm.at[j, pl.ds(h * HALF_K, HALF_K),
                            pl.ds(half * HALF_I, HALF_I)]

        cps = {0: pltpu.async_copy(chunk_src(0), rv0, sem0)}
        zero = jnp.zeros((16,), jnp.float32)
        accs = (zero, zero, zero, zero)
        for m in range(4):
            if m < 3:
                nb = (m + 1) % 2
                cps[m + 1] = pltpu.async_copy(chunk_src(m + 1), rvs[nb], sems[nb])
            cps[m].wait()
            rv = rvs[m % 2]
            h, half = divmod(m, 2)
            for kp in range(HALF_K):
                def mac(i, a, _kp=kp, _ko=h * HALF_K + kp,
                        _off=half * HALF_I, _rv=rv):
                    return tuple(
                        a[u] + (_rv[_kp, pl.ds(i + u * 16, 16)]
                                * o_v[_ko, pl.ds(_off + i + u * 16, 16)])
                        for u in range(4)
                    )
                accs = plsc.parallel_loop(0, HALF_I, 64, unroll=2,
                                          carry=accs)(mac)
        vtot = (accs[0] + accs[1]) + (accs[2] + accs[3])
        vout_v[r] = jnp.full((16,), jnp.sum(vtot), jnp.float32)

    pltpu.sync_copy(vout_v, out_hbm.at[wid])


def kernel(t, x, w, d, s):
    wt = jnp.transpose(w, (0, 2, 1))
    x2 = x.reshape(1, IN_N)
    d2 = d.reshape(DELAYS, 1)
    t2 = jnp.asarray(t, jnp.float32).reshape(1, 1)
    s2 = s.reshape(OUT_N, 1)

    o_plane = pl.pallas_call(
        _o_body,
        in_specs=[
            pl.BlockSpec((1, 1), lambda: (0, 0)),
            pl.BlockSpec((1, IN_N), lambda: (0, 0)),
            pl.BlockSpec((DELAYS, 1), lambda: (0, 0)),
        ],
        out_specs=pl.BlockSpec((DELAYS, IN_N), lambda: (0, 0)),
        out_shape=jax.ShapeDtypeStruct((DELAYS, IN_N), jnp.float32),
    )(t2, x2, d2)

    out_tc = pl.pallas_call(
        _tc_body,
        grid=(NSTEP_TC,),
        in_specs=[
            pl.BlockSpec((1, 1), lambda c: (0, 0)),
            pl.BlockSpec((BJ, 1), lambda c: (c, 0)),
            pl.BlockSpec((DELAYS, IN_N), lambda c: (0, 0)),
            pl.BlockSpec((BJ, DELAYS, IN_N), lambda c: (c, 0, 0)),
        ],
        out_specs=pl.BlockSpec((BJ, 1), lambda c: (c, 0)),
        out_shape=jax.ShapeDtypeStruct((TC_ROWS, 1), jnp.float32),
    )(t2, s2, o_plane, wt)

    mesh = plsc.VectorSubcoreMesh(core_axis_name="c", subcore_axis_name="s")
    sck = pl.kernel(
        _sc_body,
        mesh=mesh,
        out_type=jax.ShapeDtypeStruct((NW, ROWS_PER_W, 16), jnp.float32),
        scratch_types=[
            pltpu.VMEM((DELAYS, IN_N), jnp.float32),
            pltpu.VMEM((HALF_K, HALF_I), jnp.float32),
            pltpu.VMEM((HALF_K, HALF_I), jnp.float32),
            pltpu.VMEM((ROWS_PER_W, 16), jnp.float32),
            pltpu.SemaphoreType.DMA,
            pltpu.SemaphoreType.DMA,
        ],
        compiler_params=pltpu.CompilerParams(use_tc_tiling_on_sc=True,
                                             needs_layout_passes=False),
    )
    v_pad = sck(wt, o_plane)
    v_sc = v_pad[:, :, 0].reshape(SC_ROWS, 1)

    out_sc = pl.pallas_call(
        _sel_body,
        in_specs=[
            pl.BlockSpec((1, 1), lambda: (0, 0)),
            pl.BlockSpec((SC_ROWS, 1), lambda: (0, 0)),
            pl.BlockSpec((SC_ROWS, 1), lambda: (0, 0)),
        ],
        out_specs=pl.BlockSpec((SC_ROWS, 1), lambda: (0, 0)),
        out_shape=jax.ShapeDtypeStruct((SC_ROWS, 1), jnp.float32),
    )(t2, s2[TC_ROWS:], v_sc)

    return jnp.concatenate([out_tc.reshape(TC_ROWS), out_sc.reshape(SC_ROWS)])


# speedup vs baseline: 1.9245x; 1.3100x over previous
import jax
import jax.numpy as jnp
from jax.experimental import pallas as pl
from jax.experimental.pallas import tpu as pltpu

IN_N = 4096
OUT_N = 1024
DELAYS = 16
V_TH = 1.0
TAU = 5.0

BJ = 32
NSTEP = OUT_N // BJ


def _body(t_ref, x_ref, d_ref, s_ref, w_ref, out_ref, o_ref):
    tval = t_ref[0, 0]

    @pl.when(pl.program_id(0) == 0)
    def _():
        xx = x_ref[...]
        tt = tval - xx - d_ref[...]
        mask = jnp.logical_and(xx != -1.0, tt >= 0.0)
        o_ref[...] = jnp.where(mask, tt * jnp.exp(1.0 - tt / TAU) / TAU, 0.0)

    prod = w_ref[...] * o_ref[...][None]
    v = jnp.sum(prod, axis=(1, 2))
    s_old = s_ref[...]
    fire = jnp.logical_and(s_old < 0.0, v[:, None] >= V_TH)
    out_ref[...] = jnp.where(fire, tval, s_old)


def kernel(t, x, w, d, s):
    wt = jnp.transpose(w, (0, 2, 1))
    x2 = x.reshape(1, IN_N)
    d2 = d.reshape(DELAYS, 1)
    t2 = jnp.asarray(t, jnp.float32).reshape(1, 1)
    s2 = s.reshape(OUT_N, 1)

    out = pl.pallas_call(
        _body,
        grid=(NSTEP,),
        in_specs=[
            pl.BlockSpec((1, 1), lambda c: (0, 0)),
            pl.BlockSpec((1, IN_N), lambda c: (0, 0)),
            pl.BlockSpec((DELAYS, 1), lambda c: (0, 0)),
            pl.BlockSpec((BJ, 1), lambda c: (c, 0)),
            pl.BlockSpec((BJ, DELAYS, IN_N), lambda c: (c, 0, 0)),
        ],
        out_specs=pl.BlockSpec((BJ, 1), lambda c: (c, 0)),
        out_shape=jax.ShapeDtypeStruct((OUT_N, 1), jnp.float32),
        scratch_shapes=[pltpu.VMEM((DELAYS, IN_N), jnp.float32)],
    )(t2, x2, d2, s2, wt)

    return out.reshape(OUT_N)
